# baseline (speedup 1.0000x reference)
import functools

import jax
import jax.numpy as jnp
import numpy as np
from jax.experimental import pallas as pl
from jax.experimental.pallas import tpu as pltpu
from jax.sharding import Mesh, PartitionSpec as P

_TM = 512
_TH = 2048

_TRANS = (((1,), (1,)), ((), ()))


def _ffn_kernel(x_ref, w1_ref, w2_ref, o_ref):
    j = pl.program_id(1)
    nh = pl.num_programs(1)
    xb = x_ref[...].astype(jnp.bfloat16)
    h = jax.lax.dot_general(
        xb, w1_ref[...], _TRANS, preferred_element_type=jnp.float32,
    ).astype(jnp.bfloat16)
    hf = h.astype(jnp.float32)
    g = (0.5 * hf * (1.0 + jax.lax.erf(hf * np.float32(1.0 / np.sqrt(2.0))))
         ).astype(jnp.bfloat16)
    part = jax.lax.dot_general(
        g, w2_ref[...], _TRANS, preferred_element_type=jnp.float32,
    )

    @pl.when(j == 0)
    def _init():
        o_ref[...] = part

    @pl.when(j > 0)
    def _acc():
        o_ref[...] += part

    @pl.when(j == nh - 1)
    def _round():
        o_ref[...] = o_ref[...].astype(jnp.bfloat16).astype(jnp.float32)


def _ffn(xf, w1, w2):
    n, d = xf.shape
    hd = w1.shape[0]
    return pl.pallas_call(
        _ffn_kernel,
        grid=(n // _TM, hd // _TH),
        in_specs=[
            pl.BlockSpec((_TM, d), lambda i, j: (i, 0)),
            pl.BlockSpec((_TH, d), lambda i, j: (j, 0)),
            pl.BlockSpec((d, _TH), lambda i, j: (0, j)),
        ],
        out_specs=pl.BlockSpec((_TM, d), lambda i, j: (i, 0)),
        out_shape=jax.ShapeDtypeStruct((n, d), jnp.float32),
        compiler_params=pltpu.CompilerParams(
            dimension_semantics=("parallel", "arbitrary"),
        ),
    )(xf, w1, w2)


def kernel(x, Wr, W1, W2):
    B, T, D = x.shape
    N = B * T
    xf = x.reshape(N, D)
    w1 = W1.astype(jnp.bfloat16)
    w2 = W2.astype(jnp.bfloat16)
    devs = jax.devices()
    nd = 2 if len(devs) >= 2 and N % (2 * _TM) == 0 else 1
    mesh = Mesh(np.array(devs[:nd]), ("t",))
    fn = jax.shard_map(
        _ffn, mesh=mesh,
        in_specs=(P("t", None), P(None, None), P(None, None)),
        out_specs=P("t", None), check_vma=False,
    )
    return fn(xf, w1, w2).reshape(B, T, D)


# speedup vs baseline: 1.0232x; 1.0232x over previous
import functools

import jax
import jax.numpy as jnp
import numpy as np
from jax.experimental import pallas as pl
from jax.experimental.pallas import tpu as pltpu
from jax.sharding import Mesh, PartitionSpec as P

_TM = 512
_TH = 2048

_TRANS = (((1,), (1,)), ((), ()))


def _ffn_kernel(x_ref, w1_ref, w2_ref, o_ref):
    j = pl.program_id(1)
    nh = pl.num_programs(1)
    xb = x_ref[...].astype(jnp.bfloat16)
    h = jax.lax.dot_general(
        xb, w1_ref[...], _TRANS, preferred_element_type=jnp.float32,
    ).astype(jnp.bfloat16)
    hf = h.astype(jnp.float32)
    g = (0.5 * hf * (1.0 + jax.lax.erf(hf * np.float32(1.0 / np.sqrt(2.0))))
         ).astype(jnp.bfloat16)
    part = jax.lax.dot_general(
        g, w2_ref[...], _TRANS, preferred_element_type=jnp.float32,
    )

    @pl.when(j == 0)
    def _init():
        o_ref[...] = part

    @pl.when(j > 0)
    def _acc():
        o_ref[...] += part

    @pl.when(j == nh - 1)
    def _round():
        o_ref[...] = o_ref[...].astype(jnp.bfloat16).astype(jnp.float32)


def _ffn(xf, W1, W2):
    w1 = W1.astype(jnp.bfloat16)
    w2 = W2.astype(jnp.bfloat16)
    n, d = xf.shape
    hd = w1.shape[0]
    return pl.pallas_call(
        _ffn_kernel,
        grid=(n // _TM, hd // _TH),
        in_specs=[
            pl.BlockSpec((_TM, d), lambda i, j: (i, 0)),
            pl.BlockSpec((_TH, d), lambda i, j: (j, 0)),
            pl.BlockSpec((d, _TH), lambda i, j: (0, j)),
        ],
        out_specs=pl.BlockSpec((_TM, d), lambda i, j: (i, 0)),
        out_shape=jax.ShapeDtypeStruct((n, d), jnp.float32),
        compiler_params=pltpu.CompilerParams(
            dimension_semantics=("parallel", "arbitrary"),
        ),
    )(xf, w1, w2)


def kernel(x, Wr, W1, W2):
    B, T, D = x.shape
    N = B * T
    xf = x.reshape(N, D)
    devs = jax.devices()
    nd = 2 if len(devs) >= 2 and N % (2 * _TM) == 0 else 1
    mesh = Mesh(np.array(devs[:nd]), ("t",))
    fn = jax.shard_map(
        _ffn, mesh=mesh,
        in_specs=(P("t", None), P(None, None), P(None, None)),
        out_specs=P("t", None), check_vma=False,
    )
    return fn(xf, W1, W2).reshape(B, T, D)


# speedup vs baseline: 1.0828x; 1.0582x over previous
import jax
import jax.numpy as jnp
import numpy as np
from jax.experimental import pallas as pl
from jax.experimental.pallas import tpu as pltpu

_TM = 1024
_TH = 1024

_TRANS = (((1,), (1,)), ((), ()))


def _ffn_kernel(x_ref, w1_ref, w2_ref, o_ref):
    j = pl.program_id(1)
    nh = pl.num_programs(1)
    h = jax.lax.dot_general(
        x_ref[...], w1_ref[...], _TRANS, preferred_element_type=jnp.float32,
    ).astype(jnp.bfloat16)
    hf = h.astype(jnp.float32)
    g = (0.5 * hf * (1.0 + jax.lax.erf(hf * np.float32(1.0 / np.sqrt(2.0))))
         ).astype(jnp.bfloat16)
    part = jax.lax.dot_general(
        g, w2_ref[...], _TRANS, preferred_element_type=jnp.float32,
    )

    @pl.when(j == 0)
    def _init():
        o_ref[...] = part

    @pl.when(j > 0)
    def _acc():
        o_ref[...] += part

    @pl.when(j == nh - 1)
    def _round():
        o_ref[...] = o_ref[...].astype(jnp.bfloat16).astype(jnp.float32)


def kernel(x, Wr, W1, W2):
    B, T, D = x.shape
    N = B * T
    H = W1.shape[0]
    xf = x.reshape(N, D).astype(jnp.bfloat16)
    w1 = W1.astype(jnp.bfloat16)
    w2 = W2.astype(jnp.bfloat16)
    out = pl.pallas_call(
        _ffn_kernel,
        grid=(N // _TM, H // _TH),
        in_specs=[
            pl.BlockSpec((_TM, D), lambda i, j: (i, 0)),
            pl.BlockSpec((_TH, D), lambda i, j: (j, 0)),
            pl.BlockSpec((D, _TH), lambda i, j: (0, j)),
        ],
        out_specs=pl.BlockSpec((_TM, D), lambda i, j: (i, 0)),
        out_shape=jax.ShapeDtypeStruct((N, D), jnp.float32),
        compiler_params=pltpu.CompilerParams(
            dimension_semantics=("parallel", "arbitrary"),
        ),
    )(xf, w1, w2)
    return out.reshape(B, T, D)


# speedup vs baseline: 1.1269x; 1.0407x over previous
import jax
import jax.numpy as jnp
import numpy as np
from jax.experimental import pallas as pl
from jax.experimental.pallas import tpu as pltpu

_TM = 512
_TH = 2048

_TRANS = (((1,), (1,)), ((), ()))


def _ffn_kernel(x_ref, w1_ref, w2_ref, o_ref):
    j = pl.program_id(1)
    nh = pl.num_programs(1)
    h = jax.lax.dot_general(
        x_ref[...], w1_ref[...], _TRANS, preferred_element_type=jnp.float32,
    ).astype(jnp.bfloat16)
    hf = h.astype(jnp.float32)
    g = (0.5 * hf * (1.0 + jax.lax.erf(hf * np.float32(1.0 / np.sqrt(2.0))))
         ).astype(jnp.bfloat16)
    part = jax.lax.dot_general(
        g, w2_ref[...], _TRANS, preferred_element_type=jnp.float32,
    )

    @pl.when(j == 0)
    def _init():
        o_ref[...] = part

    @pl.when(j > 0)
    def _acc():
        o_ref[...] += part

    @pl.when(j == nh - 1)
    def _round():
        o_ref[...] = o_ref[...].astype(jnp.bfloat16).astype(jnp.float32)


def kernel(x, Wr, W1, W2):
    B, T, D = x.shape
    N = B * T
    H = W1.shape[0]
    xf = x.reshape(N, D).astype(jnp.bfloat16)
    w1 = W1.astype(jnp.bfloat16)
    w2 = W2.astype(jnp.bfloat16)
    out = pl.pallas_call(
        _ffn_kernel,
        grid=(N // _TM, H // _TH),
        in_specs=[
            pl.BlockSpec((_TM, D), lambda i, j: (i, 0)),
            pl.BlockSpec((_TH, D), lambda i, j: (j, 0)),
            pl.BlockSpec((D, _TH), lambda i, j: (0, j)),
        ],
        out_specs=pl.BlockSpec((_TM, D), lambda i, j: (i, 0)),
        out_shape=jax.ShapeDtypeStruct((N, D), jnp.float32),
        compiler_params=pltpu.CompilerParams(
            dimension_semantics=("parallel", "arbitrary"),
        ),
    )(xf, w1, w2)
    return out.reshape(B, T, D)


# speedup vs baseline: 1.1913x; 1.0572x over previous
import jax
import jax.numpy as jnp
import numpy as np
from jax.experimental import pallas as pl
from jax.experimental.pallas import tpu as pltpu

_TM = 512
_TH = 2048

_TRANS = (((1,), (1,)), ((), ()))


def _ffn_kernel(x_ref, w1_ref, w2_ref, o_ref):
    j = pl.program_id(1)
    nh = pl.num_programs(1)
    xb = x_ref[...].astype(jnp.bfloat16)
    h = jax.lax.dot_general(
        xb, w1_ref[...], _TRANS, preferred_element_type=jnp.float32,
    ).astype(jnp.bfloat16)
    hf = h.astype(jnp.float32)
    g = (0.5 * hf * (1.0 + jax.lax.erf(hf * np.float32(1.0 / np.sqrt(2.0))))
         ).astype(jnp.bfloat16)
    part = jax.lax.dot_general(
        g, w2_ref[...], _TRANS, preferred_element_type=jnp.float32,
    )

    @pl.when(j == 0)
    def _init():
        o_ref[...] = part

    @pl.when((j > 0) & (j < nh - 1))
    def _acc():
        o_ref[...] += part

    @pl.when(j == nh - 1)
    def _last():
        o_ref[...] = (o_ref[...] + part).astype(jnp.bfloat16).astype(jnp.float32)


def kernel(x, Wr, W1, W2):
    B, T, D = x.shape
    N = B * T
    H = W1.shape[0]
    xf = x.reshape(N, D)
    w1 = W1.astype(jnp.bfloat16)
    w2 = W2.astype(jnp.bfloat16)
    out = pl.pallas_call(
        _ffn_kernel,
        grid=(N // _TM, H // _TH),
        in_specs=[
            pl.BlockSpec((_TM, D), lambda i, j: (i, 0)),
            pl.BlockSpec((_TH, D), lambda i, j: (j, 0)),
            pl.BlockSpec((D, _TH), lambda i, j: (0, j)),
        ],
        out_specs=pl.BlockSpec((_TM, D), lambda i, j: (i, 0)),
        out_shape=jax.ShapeDtypeStruct((N, D), jnp.float32),
        compiler_params=pltpu.CompilerParams(
            dimension_semantics=("parallel", "arbitrary"),
        ),
    )(xf, w1, w2)
    return out.reshape(B, T, D)
